# baseline (speedup 1.0000x reference)
import jax
import jax.numpy as jnp
from jax import lax
from jax.experimental import pallas as pl
from jax.experimental.pallas import tpu as pltpu
from jax.experimental.pallas import tpu_sc as plsc

EMB = 64
SCALE = 8.0

NW = 32
GI = 2
GSZ = GI * 50
NGRP = 256
NSLOT = 4
I_PER_W = NGRP * GI


def _emb_body(xr_hbm, table_hbm, out_hbm, idx_v, grows_v, orows_v, gsem, osem):
    c = lax.axis_index("c")
    s = lax.axis_index("s")
    wid = s * 2 + c
    ibase = wid * I_PER_W

    pltpu.sync_copy(xr_hbm.at[wid], idx_v)

    def fire_gather(g, slot):
        pltpu.make_async_copy(
            table_hbm.at[idx_v.at[g]],
            grows_v.at[slot],
            gsem.at[slot],
        ).start()

    def wait_gather(slot):
        pltpu.make_async_copy(
            table_hbm.at[idx_v.at[0]],
            grows_v.at[slot],
            gsem.at[slot],
        ).wait()

    def scale_slot(slot):
        def body(j, carry):
            for i2 in range(GI):
                r = i2 * 50 + j
                for cc in range(4):
                    sl = pl.ds(cc * 16, 16)
                    orows_v[slot, i2, j, sl] = grows_v[slot, r, sl] * SCALE
            return carry

        lax.fori_loop(0, 50, body, 0)

    def out_desc(g, slot):
        return pltpu.make_async_copy(
            orows_v.at[slot],
            out_hbm.at[pl.ds(ibase + g * GI, GI)],
            osem.at[slot],
        )

    def consume(g, slot, wait_out):
        wait_gather(slot)
        if wait_out:
            out_desc(g, slot).wait()
        scale_slot(slot)
        out_desc(g, slot).start()

    fire_gather(0, 0)
    fire_gather(1, 1)

    for g in range(6):
        fire_gather(g + 2, (g + 2) % 4)
        consume(g, g % 4, wait_out=(g >= 4))

    def main_blk(i, carry):
        g0 = 6 + i * 4
        for db in range(4):
            g = g0 + db
            slot = (2 + db) % 4
            fire_gather(g + 2, db)
            consume(g, slot, wait_out=True)
        return carry

    lax.fori_loop(0, (NGRP - 8) // 4, main_blk, 0)

    consume(NGRP - 2, 2, wait_out=True)
    consume(NGRP - 1, 3, wait_out=True)

    for slot in range(NSLOT):
        out_desc(0, slot).wait()


def kernel(x, table):
    xr = x.astype(jnp.int32).reshape(NW, NGRP, GSZ)
    mesh = plsc.VectorSubcoreMesh(core_axis_name="c", subcore_axis_name="s")
    out = pl.kernel(
        _emb_body,
        out_type=jax.ShapeDtypeStruct((x.shape[0], x.shape[1], EMB), jnp.float32),
        mesh=mesh,
        compiler_params=pltpu.CompilerParams(use_tc_tiling_on_sc=False),
        scratch_types=[
            pltpu.VMEM((NGRP, GSZ), jnp.int32),
            pltpu.VMEM((NSLOT, GSZ, EMB), jnp.float32),
            pltpu.VMEM((NSLOT, GI, 50, EMB), jnp.float32),
            pltpu.SemaphoreType.DMA((NSLOT,)),
            pltpu.SemaphoreType.DMA((NSLOT,)),
        ],
    )(xr, table)
    return out
